# baseline (speedup 1.0000x reference)
import functools

import jax
import jax.numpy as jnp
from jax import lax
from jax.experimental import pallas as pl
from jax.experimental.pallas import tpu as pltpu
from jax.experimental.pallas import tpu_sc as plsc

N, K, D = 10000, 32, 128
NC, NS, L = 2, 16, 16
N_PAD = 10240
HALF = N_PAD // NC
PAIRS = HALF // 2
P = 128
CH_N = 2 * P
NCH = PAIRS // P
NR = D // NS


def _sc_body(pidx_ref, xt_ref, agg_ref, tbl_v, stg_v, idx_v, out0_v, out1_v,
             si0, si1, so0, so1):
    outs = (out0_v, out1_v)
    c = lax.axis_index("c")
    s = lax.axis_index("s")
    pbase = c * PAIRS

    iota = lax.broadcasted_iota(jnp.int32, (L,), 0)
    iota8 = iota * NR

    for r in range(NR):
        pltpu.sync_copy(xt_ref.at[s * NR + r], stg_v)

        @plsc.parallel_loop(0, N_PAD // L, step=1, unroll=4)
        def row_scatter(v):
            vals = stg_v[pl.ds(v * L, L)]
            plsc.store_scatter(tbl_v, [iota8 + (v * (L * NR) + r)], vals)
    rowv = iota & 7
    perms = [2 * m + jnp.where(iota >= 8, 1, 0) for m in range(8)]
    sbase = rowv * CH_N + jnp.where(iota >= 8, 1, 0)

    isems = (si0, si1)
    osems = (so0, so1)

    def idx_copy(ch, b):
        return pltpu.make_async_copy(
            pidx_ref.at[pl.ds(pbase + ch * P, P)], idx_v.at[b], isems[b])

    def out_copies(ch, b):
        node0 = c * HALF + ch * CH_N
        return [
            pltpu.make_async_copy(
                outs[b].at[pl.ds(r * CH_N, CH_N)],
                agg_ref.at[s * NR + r, pl.ds(node0, CH_N)],
                osems[b])
            for r in range(NR)
        ]

    idx_copy(0, 0).start()
    idx_copy(1, 1).start()

    def chunk(cc, ch, b):
        idx_copy(ch, b).wait()

        @pl.when(cc > 0)
        def _():
            for cp in out_copies(ch, b):
                cp.wait()

        @plsc.parallel_loop(0, P, step=1, unroll=8)
        def pair_body(p):
            accs = [None] * 4
            for kg in range(4):
                vkg = idx_v[b, p, pl.ds(kg * 16, 16)]
                for m in range(8):
                    t = jnp.take_along_axis(vkg, perms[m], axis=0)
                    val = plsc.load_gather(tbl_v, [t + rowv])
                    accs[kg] = val if accs[kg] is None else accs[kg] + val
            acc = (accs[0] + accs[1]) + (accs[2] + accs[3])
            plsc.store_scatter(outs[b], [2 * p + sbase], acc)

        for cp in out_copies(ch, b):
            cp.start()

        @pl.when(ch + 2 < NCH)
        def _():
            idx_copy(ch + 2, b).start()

    def loop_body(cc, carry):
        chunk(cc, 2 * cc, 0)
        chunk(cc, 2 * cc + 1, 1)
        return carry

    lax.fori_loop(0, NCH // 2, loop_body, 0)
    for cp in out_copies(NCH - 2, 0):
        cp.wait()
    for cp in out_copies(NCH - 1, 1):
        cp.wait()


@functools.cache
def _gather_sum_kernel():
    return pl.kernel(
        _sc_body,
        out_type=jax.ShapeDtypeStruct((D, N_PAD), jnp.float32),
        mesh=plsc.VectorSubcoreMesh(core_axis_name="c", subcore_axis_name="s"),
        compiler_params=pltpu.CompilerParams(needs_layout_passes=False),
        scratch_types=[
            pltpu.VMEM((N_PAD * NR,), jnp.float32),
            pltpu.VMEM((N_PAD,), jnp.float32),
            pltpu.VMEM((2, P, 2 * K), jnp.int32),
            pltpu.VMEM((CH_N * NR,), jnp.float32),
            pltpu.VMEM((CH_N * NR,), jnp.float32),
            pltpu.SemaphoreType.DMA,
            pltpu.SemaphoreType.DMA,
            pltpu.SemaphoreType.DMA,
            pltpu.SemaphoreType.DMA,
        ],
    )


def _gather_sum(pidx, xb):
    return _gather_sum_kernel()(pidx, xb)


def _to_blocked(xt):
    return xt.reshape(NS, NR, N_PAD).transpose(0, 2, 1).reshape(
        NS, N_PAD * NR)


def _dense_t(xt_ref, gt_ref, w1_ref, w2_ref, b_ref):
    cd = (((0,), (0,)), ((), ()))
    return (
        lax.dot_general(w1_ref[...], xt_ref[...], cd, precision="highest",
                        preferred_element_type=jnp.float32)
        + lax.dot_general(w2_ref[...], gt_ref[...] * (1.0 / K), cd,
                          precision="highest",
                          preferred_element_type=jnp.float32)
        + b_ref[...]
    )


def _tc_body(xt_ref, gt_ref, w1_ref, w2_ref, b_ref, out_ref, *, relu):
    h = _dense_t(xt_ref, gt_ref, w1_ref, w2_ref, b_ref)
    out_ref[...] = jnp.maximum(h, 0.0) if relu else h


def _tc_body_resid(xt_ref, gt_ref, w1_ref, w2_ref, b_ref, t_ref, out_ref):
    h = _dense_t(xt_ref, gt_ref, w1_ref, w2_ref, b_ref)
    out_ref[...] = (t_ref[...] + jnp.maximum(h, 0.0)) * 0.5


_BLK = 1024


def _combine(xt, gt, w1, w2, b, relu, temp=None):
    col_spec = pl.BlockSpec((D, _BLK), lambda i: (0, i))
    w_spec = pl.BlockSpec((D, D), lambda i: (0, 0))
    b_spec = pl.BlockSpec((D, 1), lambda i: (0, 0))
    out_shape = jax.ShapeDtypeStruct((D, N_PAD), jnp.float32)
    if temp is None:
        return pl.pallas_call(
            functools.partial(_tc_body, relu=relu),
            grid=(N_PAD // _BLK,),
            in_specs=[col_spec, col_spec, w_spec, w_spec, b_spec],
            out_specs=col_spec,
            out_shape=out_shape,
        )(xt, gt, w1, w2, b.reshape(D, 1))
    return pl.pallas_call(
        _tc_body_resid,
        grid=(N_PAD // _BLK,),
        in_specs=[col_spec, col_spec, w_spec, w_spec, b_spec, col_spec],
        out_specs=col_spec,
        out_shape=out_shape,
    )(xt, gt, w1, w2, b.reshape(D, 1), temp)


def kernel(neighbours, shape_features, W1s, W2s, bs, W1_out, W2_out, b_out):
    nbr = jnp.asarray(neighbours, jnp.int32)
    nbr_pad = jnp.zeros((N_PAD, K), jnp.int32).at[:N].set(nbr)
    pidx = (nbr_pad * NR).reshape(N_PAD // 2, 2, K).transpose(0, 2, 1)
    pidx = pidx.reshape(N_PAD // 2, 2 * K)

    xt = jnp.zeros((D, N_PAD), jnp.float32).at[:, :N].set(shape_features.T)

    def gcn(xt, w1, w2, b, relu, temp=None):
        gt = _gather_sum(pidx, xt)
        return _combine(xt, gt, w1, w2, b, relu, temp)

    xt = gcn(xt, W1s[0], W2s[0], bs[0], True)
    for i in range(1, 12, 2):
        t = xt
        xt = gcn(xt, W1s[i], W2s[i], bs[i], True)
        xt = gcn(xt, W1s[i + 1], W2s[i + 1], bs[i + 1], True, temp=t)

    w1o = jnp.zeros((D, D), jnp.float32).at[:, :3].set(W1_out)
    w2o = jnp.zeros((D, D), jnp.float32).at[:, :3].set(W2_out)
    bo = jnp.zeros((D,), jnp.float32).at[:3].set(b_out)
    coords_t = gcn(xt, w1o, w2o, bo, False)
    return (xt.T[:N], coords_t.T[:N, :3])


# speedup vs baseline: 1.4015x; 1.4015x over previous
import functools

import jax
import jax.numpy as jnp
from jax import lax
from jax.experimental import pallas as pl
from jax.experimental.pallas import tpu as pltpu
from jax.experimental.pallas import tpu_sc as plsc

N, K, D = 10000, 32, 128
NC, NS, L = 2, 16, 16
N_PAD = 10240
HALF = N_PAD // NC
PAIRS = HALF // 2
P = 128
CH_N = 2 * P
NCH = PAIRS // P
NR = D // NS


def _sc_body(pidx_ref, xt_ref, agg_ref, tbl_v, stg_v, idx_v, out0_v, out1_v,
             si0, si1, so0, so1):
    outs = (out0_v, out1_v)
    c = lax.axis_index("c")
    s = lax.axis_index("s")
    pbase = c * PAIRS

    iota = lax.broadcasted_iota(jnp.int32, (L,), 0)
    iota8 = iota * NR

    for r in range(NR):
        pltpu.sync_copy(xt_ref.at[s * NR + r], stg_v)

        @plsc.parallel_loop(0, N_PAD // L, step=1, unroll=4)
        def row_scatter(v):
            vals = stg_v[pl.ds(v * L, L)]
            plsc.store_scatter(tbl_v, [iota8 + (v * (L * NR) + r)], vals)
    rowv = iota & 7
    perms = [2 * m + jnp.where(iota >= 8, 1, 0) for m in range(8)]
    sbase = rowv * CH_N + jnp.where(iota >= 8, 1, 0)

    isems = (si0, si1)
    osems = (so0, so1)

    def idx_copy(ch, b):
        return pltpu.make_async_copy(
            pidx_ref.at[pl.ds(pbase + ch * P, P)], idx_v.at[b], isems[b])

    def out_copies(ch, b):
        node0 = c * HALF + ch * CH_N
        return [
            pltpu.make_async_copy(
                outs[b].at[pl.ds(r * CH_N, CH_N)],
                agg_ref.at[s * NR + r, pl.ds(node0, CH_N)],
                osems[b])
            for r in range(NR)
        ]

    idx_copy(0, 0).start()
    idx_copy(1, 1).start()

    def chunk(cc, ch, b):
        idx_copy(ch, b).wait()

        @pl.when(cc > 0)
        def _():
            for cp in out_copies(ch, b):
                cp.wait()

        @plsc.parallel_loop(0, P, step=1, unroll=4)
        def pair_body(p):
            accs = [None] * 4
            for kg in range(4):
                vkg = idx_v[b, p, pl.ds(kg * 16, 16)]
                for m in range(8):
                    t = jnp.take_along_axis(vkg, perms[m], axis=0)
                    val = plsc.load_gather(tbl_v, [t + rowv])
                    accs[kg] = val if accs[kg] is None else accs[kg] + val
            acc = (accs[0] + accs[1]) + (accs[2] + accs[3])
            plsc.store_scatter(outs[b], [2 * p + sbase], acc)

        for cp in out_copies(ch, b):
            cp.start()

        @pl.when(ch + 2 < NCH)
        def _():
            idx_copy(ch + 2, b).start()

    def loop_body(cc, carry):
        chunk(cc, 2 * cc, 0)
        chunk(cc, 2 * cc + 1, 1)
        return carry

    lax.fori_loop(0, NCH // 2, loop_body, 0)
    for cp in out_copies(NCH - 2, 0):
        cp.wait()
    for cp in out_copies(NCH - 1, 1):
        cp.wait()


@functools.cache
def _gather_sum_kernel():
    return pl.kernel(
        _sc_body,
        out_type=jax.ShapeDtypeStruct((D, N_PAD), jnp.float32),
        mesh=plsc.VectorSubcoreMesh(core_axis_name="c", subcore_axis_name="s"),
        compiler_params=pltpu.CompilerParams(needs_layout_passes=False),
        scratch_types=[
            pltpu.VMEM((N_PAD * NR,), jnp.float32),
            pltpu.VMEM((N_PAD,), jnp.float32),
            pltpu.VMEM((2, P, 2 * K), jnp.int32),
            pltpu.VMEM((CH_N * NR,), jnp.float32),
            pltpu.VMEM((CH_N * NR,), jnp.float32),
            pltpu.SemaphoreType.DMA,
            pltpu.SemaphoreType.DMA,
            pltpu.SemaphoreType.DMA,
            pltpu.SemaphoreType.DMA,
        ],
    )


def _gather_sum(pidx, xt):
    return _gather_sum_kernel()(pidx, xt)


def _dense_t(xt_ref, gt_ref, w1_ref, w2_ref, b_ref):
    cd = (((0,), (0,)), ((), ()))
    return (
        lax.dot_general(w1_ref[...], xt_ref[...], cd, precision="highest",
                        preferred_element_type=jnp.float32)
        + lax.dot_general(w2_ref[...], gt_ref[...] * (1.0 / K), cd,
                          precision="highest",
                          preferred_element_type=jnp.float32)
        + b_ref[...]
    )


def _tc_body(xt_ref, gt_ref, w1_ref, w2_ref, b_ref, out_ref, *, relu):
    h = _dense_t(xt_ref, gt_ref, w1_ref, w2_ref, b_ref)
    out_ref[...] = jnp.maximum(h, 0.0) if relu else h


def _tc_body_resid(xt_ref, gt_ref, w1_ref, w2_ref, b_ref, t_ref, out_ref):
    h = _dense_t(xt_ref, gt_ref, w1_ref, w2_ref, b_ref)
    out_ref[...] = (t_ref[...] + jnp.maximum(h, 0.0)) * 0.5


_BLK = 1024


def _combine(xt, gt, w1, w2, b, relu, temp=None):
    col_spec = pl.BlockSpec((D, _BLK), lambda i: (0, i))
    w_spec = pl.BlockSpec((D, D), lambda i: (0, 0))
    b_spec = pl.BlockSpec((D, 1), lambda i: (0, 0))
    out_shape = jax.ShapeDtypeStruct((D, N_PAD), jnp.float32)
    if temp is None:
        return pl.pallas_call(
            functools.partial(_tc_body, relu=relu),
            grid=(N_PAD // _BLK,),
            in_specs=[col_spec, col_spec, w_spec, w_spec, b_spec],
            out_specs=col_spec,
            out_shape=out_shape,
        )(xt, gt, w1, w2, b.reshape(D, 1))
    return pl.pallas_call(
        _tc_body_resid,
        grid=(N_PAD // _BLK,),
        in_specs=[col_spec, col_spec, w_spec, w_spec, b_spec, col_spec],
        out_specs=col_spec,
        out_shape=out_shape,
    )(xt, gt, w1, w2, b.reshape(D, 1), temp)


def kernel(neighbours, shape_features, W1s, W2s, bs, W1_out, W2_out, b_out):
    nbr = jnp.asarray(neighbours, jnp.int32)
    nbr_pad = jnp.zeros((N_PAD, K), jnp.int32).at[:N].set(nbr)
    pidx = (nbr_pad * NR).reshape(N_PAD // 2, 2, K).transpose(0, 2, 1)
    pidx = pidx.reshape(N_PAD // 2, 2 * K)

    xt = jnp.zeros((D, N_PAD), jnp.float32).at[:, :N].set(shape_features.T)

    def gcn(xt, w1, w2, b, relu, temp=None):
        gt = _gather_sum(pidx, xt)
        return _combine(xt, gt, w1, w2, b, relu, temp)

    xt = gcn(xt, W1s[0], W2s[0], bs[0], True)
    for i in range(1, 12, 2):
        t = xt
        xt = gcn(xt, W1s[i], W2s[i], bs[i], True)
        xt = gcn(xt, W1s[i + 1], W2s[i + 1], bs[i + 1], True, temp=t)

    w1o = jnp.zeros((D, D), jnp.float32).at[:, :3].set(W1_out)
    w2o = jnp.zeros((D, D), jnp.float32).at[:, :3].set(W2_out)
    bo = jnp.zeros((D,), jnp.float32).at[:3].set(b_out)
    coords_t = gcn(xt, w1o, w2o, bo, False)
    return (xt.T[:N], coords_t.T[:N, :3])
